# Initial kernel scaffold; baseline (speedup 1.0000x reference)
import jax
import jax.numpy as jnp
from jax.experimental import pallas as pl


def kernel(H, E, ht, queries, W_fwd, b_fwd, W_back, b_back, ln_gamma, ln_beta):
    raise NotImplementedError("write your pallas kernel here")


# speedup vs baseline: 2.7673x; 2.7673x over previous
import functools

import jax
import jax.numpy as jnp
from jax import lax
from jax.experimental import pallas as pl
from jax.experimental.pallas import tpu as pltpu
from jax.experimental.pallas import tpu_sc as plsc

N_NODES = 10000
N_EDGES = 320000
D = 128

NCORES = 2
NTILES = 16
EDGES_PER_CORE = N_EDGES // NCORES
EDGES_PER_TILE = EDGES_PER_CORE // NTILES
CHUNK = 80
NCHUNKS = EDGES_PER_TILE // CHUNK
NP = 10240
ROWS_PER_TILE = NP // NTILES


def _sc_accumulate(H, E, heads, tails):
    f32 = jnp.float32

    @functools.partial(
        pl.kernel,
        mesh=plsc.VectorSubcoreMesh(core_axis_name="c", subcore_axis_name="s"),
        out_type=(
            jax.ShapeDtypeStruct((2 * NCORES, NP, D), f32),
            jax.ShapeDtypeStruct((2 * NCORES, NP, D), f32),
            jax.ShapeDtypeStruct((2 * NCORES, NP, D), f32),
        ),
        scratch_types=[
            pltpu.VMEM_SHARED((NP, D), f32),
            pltpu.VMEM((64, D), f32),
            pltpu.VMEM((CHUNK, D), f32),
            pltpu.VMEM((CHUNK,), jnp.int32),
            pltpu.VMEM((CHUNK,), jnp.int32),
            pltpu.VMEM((CHUNK, D), f32),
        ],
    )
    def sc_kernel(H_hbm, E_hbm, heads_hbm, tails_hbm, S4, T4, C4,
                  acc, zbuf, gbuf, idxg, idxs, obuf):
        cid = lax.axis_index("c")
        sid = lax.axis_index("s")
        ebase = cid * EDGES_PER_CORE + sid * EDGES_PER_TILE
        rbase = sid * ROWS_PER_TILE

        zero16 = jnp.zeros((16,), f32)
        one16 = jnp.ones((16,), f32)

        def zrow(r, carry):
            for c8 in range(D // 16):
                zbuf[r, pl.ds(c8 * 16, 16)] = zero16
            return carry
        lax.fori_loop(0, 64, zrow, 0)

        def orow(r, carry):
            for c8 in range(D // 16):
                obuf[r, pl.ds(c8 * 16, 16)] = one16
            return carry
        lax.fori_loop(0, CHUNK, orow, 0)

        for dirn in (0, 1):
            gref, sref = (heads_hbm, tails_hbm) if dirn == 0 else (tails_hbm, heads_hbm)
            for j in range(10):
                pltpu.sync_copy(zbuf, acc.at[pl.ds(rbase + j * 64, 64)])
            plsc.subcore_barrier()

            def sbody(i, carry, gref=gref, sref=sref):
                off = ebase + i * CHUNK
                pltpu.sync_copy(gref.at[pl.ds(off, CHUNK)], idxg)
                pltpu.sync_copy(sref.at[pl.ds(off, CHUNK)], idxs)
                pltpu.sync_copy(H_hbm.at[idxg], gbuf.at[pl.ds(0, CHUNK)])
                pltpu.sync_copy(gbuf.at[pl.ds(0, CHUNK)], acc.at[idxs], add=True)
                return carry
            lax.fori_loop(0, NCHUNKS, sbody, 0)
            plsc.subcore_barrier()

            plane = 2 * cid + dirn
            for j in range(8):
                pltpu.sync_copy(acc.at[pl.ds(rbase + j * 80, 80)], gbuf)
                pltpu.sync_copy(gbuf, S4.at[plane, pl.ds(rbase + j * 80, 80)])

        for dirn in (0, 1):
            sref = tails_hbm if dirn == 0 else heads_hbm
            for j in range(10):
                pltpu.sync_copy(zbuf, acc.at[pl.ds(rbase + j * 64, 64)])
            plsc.subcore_barrier()

            def tbody(i, carry, sref=sref):
                off = ebase + i * CHUNK
                pltpu.sync_copy(E_hbm.at[pl.ds(off, CHUNK)],
                                gbuf.at[pl.ds(0, CHUNK)])
                pltpu.sync_copy(sref.at[pl.ds(off, CHUNK)], idxs)
                pltpu.sync_copy(gbuf.at[pl.ds(0, CHUNK)], acc.at[idxs], add=True)
                return carry
            lax.fori_loop(0, NCHUNKS, tbody, 0)
            plsc.subcore_barrier()

            plane = 2 * cid + dirn
            for j in range(8):
                pltpu.sync_copy(acc.at[pl.ds(rbase + j * 80, 80)], gbuf)
                pltpu.sync_copy(gbuf, T4.at[plane, pl.ds(rbase + j * 80, 80)])

        for dirn in (0, 1):
            sref = tails_hbm if dirn == 0 else heads_hbm
            for j in range(10):
                pltpu.sync_copy(zbuf, acc.at[pl.ds(rbase + j * 64, 64)])
            plsc.subcore_barrier()

            def cbody(i, carry, sref=sref):
                off = ebase + i * CHUNK
                pltpu.sync_copy(sref.at[pl.ds(off, CHUNK)], idxs)
                pltpu.sync_copy(obuf, acc.at[idxs], add=True)
                return carry
            lax.fori_loop(0, NCHUNKS, cbody, 0)
            plsc.subcore_barrier()

            plane = 2 * cid + dirn
            for j in range(8):
                pltpu.sync_copy(acc.at[pl.ds(rbase + j * 80, 80)], gbuf)
                pltpu.sync_copy(gbuf, C4.at[plane, pl.ds(rbase + j * 80, 80)])

    return sc_kernel(H, E, heads, tails)


def _tc_finish_body(s4, t4, c4, h, wbig, bf, bb, gam, bet, out):
    s = s4[...]
    t = t4[...]
    c = c4[...]
    Sf = s[0] + s[2]
    Sb = s[1] + s[3]
    Tf = t[0] + t[2]
    Tb = t[1] + t[3]
    X = jnp.concatenate([Sf, Tf, Sb, Tb], axis=1)
    num = jnp.dot(X, wbig[...], preferred_element_type=jnp.float32)
    ct = c[0, :, 0:1] + c[2, :, 0:1]
    ch = c[1, :, 0:1] + c[3, :, 0:1]
    num = num + ct * bf[...] + ch * bb[...]
    agg = num / (ct + ch)
    x = jnp.where(agg >= 0, agg, 0.01 * agg) + h[...]
    mu = jnp.mean(x, axis=1, keepdims=True)
    var = jnp.mean((x - mu) ** 2, axis=1, keepdims=True)
    out[...] = (x - mu) * lax.rsqrt(var + 1e-5) * gam[...] + bet[...]


def _tc_finish(S4, T4, C4, H, Wbig, bf, bb, gam, bet):
    BR = 1024
    grid = NP // BR
    return pl.pallas_call(
        _tc_finish_body,
        grid=(grid,),
        in_specs=[
            pl.BlockSpec((2 * NCORES, BR, D), lambda i: (0, i, 0)),
            pl.BlockSpec((2 * NCORES, BR, D), lambda i: (0, i, 0)),
            pl.BlockSpec((2 * NCORES, BR, D), lambda i: (0, i, 0)),
            pl.BlockSpec((BR, D), lambda i: (i, 0)),
            pl.BlockSpec((4 * D, D), lambda i: (0, 0)),
            pl.BlockSpec((1, D), lambda i: (0, 0)),
            pl.BlockSpec((1, D), lambda i: (0, 0)),
            pl.BlockSpec((1, D), lambda i: (0, 0)),
            pl.BlockSpec((1, D), lambda i: (0, 0)),
        ],
        out_specs=pl.BlockSpec((BR, D), lambda i: (i, 0)),
        out_shape=jax.ShapeDtypeStruct((NP, D), jnp.float32),
    )(S4, T4, C4, H, Wbig, bf, bb, gam, bet)


def kernel(H, E, ht, queries, W_fwd, b_fwd, W_back, b_back, ln_gamma, ln_beta):
    ht32 = ht.astype(jnp.int32)
    S4, T4, C4 = _sc_accumulate(H, E, ht32[:, 0], ht32[:, 1])
    Wbig = jnp.concatenate(
        [W_fwd[:, :D].T, W_fwd[:, D:].T, W_back[:, :D].T, W_back[:, D:].T],
        axis=0)
    Hp = jnp.pad(H, ((0, NP - N_NODES), (0, 0)))
    out = _tc_finish(S4, T4, C4, Hp, Wbig,
                     b_fwd.reshape(1, D), b_back.reshape(1, D),
                     ln_gamma.reshape(1, D), ln_beta.reshape(1, D))
    return out[:N_NODES]


# speedup vs baseline: 4.1966x; 1.5165x over previous
import functools

import jax
import jax.numpy as jnp
from jax import lax
from jax.experimental import pallas as pl
from jax.experimental.pallas import tpu as pltpu
from jax.experimental.pallas import tpu_sc as plsc

N_NODES = 10000
N_EDGES = 320000
D = 128

NCORES = 2
NTILES = 16
EDGES_PER_CORE = N_EDGES // NCORES
EDGES_PER_TILE = EDGES_PER_CORE // NTILES
CHUNK = 80
NCHUNKS = EDGES_PER_TILE // CHUNK
NP = 10240
ROWS_PER_TILE = NP // NTILES


def _sc_accumulate(H, E, heads, tails):
    f32 = jnp.float32

    @functools.partial(
        pl.kernel,
        mesh=plsc.VectorSubcoreMesh(core_axis_name="c", subcore_axis_name="s"),
        out_type=(
            jax.ShapeDtypeStruct((2 * NCORES, NP, D), f32),
            jax.ShapeDtypeStruct((2 * NCORES, NP, D), f32),
            jax.ShapeDtypeStruct((2 * NCORES, NP, D), f32),
        ),
        scratch_types=[
            pltpu.VMEM_SHARED((NP, D), f32),
            pltpu.VMEM((2 * CHUNK, D), f32),
            pltpu.VMEM((CHUNK,), jnp.int32),
            pltpu.VMEM((CHUNK,), jnp.int32),
            pltpu.VMEM((CHUNK,), jnp.int32),
            pltpu.VMEM((CHUNK,), jnp.int32),
            pltpu.SemaphoreType.DMA,
            pltpu.SemaphoreType.DMA,
        ],
    )
    def sc_kernel(H_hbm, E_hbm, heads_hbm, tails_hbm, S4, T4, C4,
                  acc, gbuf, idxg0, idxg1, idxs0, idxs1, sem0, sem1):
        cid = lax.axis_index("c")
        sid = lax.axis_index("s")
        ebase = cid * EDGES_PER_CORE + sid * EDGES_PER_TILE
        rbase = sid * ROWS_PER_TILE
        NPAIR = (NCHUNKS - 1) // 2

        b0 = gbuf.at[pl.ds(0, CHUNK)]
        b1 = gbuf.at[pl.ds(CHUNK, CHUNK)]

        def ioff(c):
            return pl.ds(ebase + c * CHUNK, CHUNK)

        def ild(ref, c, dst, sem):
            pltpu.async_copy(ref.at[ioff(c)], dst, sem)
            pltpu.make_async_copy(ref.at[ioff(c)], dst, sem).wait()

        def fill(buf, val):
            v = jnp.full((16,), val, f32)
            def row(r, carry):
                for c8 in range(D // 16):
                    buf[r, pl.ds(c8 * 16, 16)] = v
                return carry
            lax.fori_loop(0, CHUNK, row, 0)

        def zero_acc():
            fill(b1, 0.0)
            for j in range(8):
                pltpu.sync_copy(b1, acc.at[pl.ds(rbase + j * CHUNK, CHUNK)])

        def copy_out(dst, plane):
            for j in range(8):
                pltpu.sync_copy(acc.at[pl.ds(rbase + j * CHUNK, CHUNK)], b0)
                pltpu.sync_copy(b0, dst.at[plane, pl.ds(rbase + j * CHUNK, CHUNK)])

        def s_phase(gref, sref):
            zero_acc()
            plsc.subcore_barrier()
            ild(gref, 0, idxg0, sem0)
            ild(sref, 0, idxs0, sem0)
            ild(gref, 1, idxg1, sem1)
            ild(sref, 1, idxs1, sem1)
            pltpu.async_copy(H_hbm.at[idxg0], b0, sem0)

            def pair(k, carry):
                i = 2 * k
                pltpu.async_copy(H_hbm.at[idxg1], b1, sem1)
                pltpu.make_async_copy(H_hbm.at[idxg0], b0, sem0).wait()
                pltpu.sync_copy(b0, acc.at[idxs0], add=True)
                ild(gref, i + 2, idxg0, sem0)
                ild(sref, i + 2, idxs0, sem0)
                pltpu.async_copy(H_hbm.at[idxg0], b0, sem0)
                pltpu.make_async_copy(H_hbm.at[idxg1], b1, sem1).wait()
                pltpu.sync_copy(b1, acc.at[idxs1], add=True)
                c3 = jnp.minimum(i + 3, NCHUNKS - 1)
                ild(gref, c3, idxg1, sem1)
                ild(sref, c3, idxs1, sem1)
                return carry
            lax.fori_loop(0, NPAIR, pair, 0)
            pltpu.make_async_copy(H_hbm.at[idxg0], b0, sem0).wait()
            pltpu.sync_copy(b0, acc.at[idxs0], add=True)
            plsc.subcore_barrier()

        def t_phase(sref):
            zero_acc()
            plsc.subcore_barrier()
            ild(sref, 0, idxs0, sem0)
            ild(sref, 1, idxs1, sem1)
            pltpu.async_copy(E_hbm.at[ioff(0)], b0, sem0)

            def pair(k, carry):
                i = 2 * k
                pltpu.async_copy(E_hbm.at[ioff(i + 1)], b1, sem1)
                pltpu.make_async_copy(E_hbm.at[ioff(i)], b0, sem0).wait()
                pltpu.sync_copy(b0, acc.at[idxs0], add=True)
                ild(sref, i + 2, idxs0, sem0)
                pltpu.async_copy(E_hbm.at[ioff(i + 2)], b0, sem0)
                pltpu.make_async_copy(E_hbm.at[ioff(i + 1)], b1, sem1).wait()
                pltpu.sync_copy(b1, acc.at[idxs1], add=True)
                c3 = jnp.minimum(i + 3, NCHUNKS - 1)
                ild(sref, c3, idxs1, sem1)
                return carry
            lax.fori_loop(0, NPAIR, pair, 0)
            pltpu.make_async_copy(E_hbm.at[ioff(NCHUNKS - 1)], b0, sem0).wait()
            pltpu.sync_copy(b0, acc.at[idxs0], add=True)
            plsc.subcore_barrier()

        def c_phase(sref):
            zero_acc()
            plsc.subcore_barrier()
            fill(b0, 1.0)
            ild(sref, 0, idxs0, sem0)
            ild(sref, 1, idxs1, sem1)

            def pair(k, carry):
                i = 2 * k
                pltpu.async_copy(b0, acc.at[idxs0], sem0, add=True)
                pltpu.async_copy(b0, acc.at[idxs1], sem1, add=True)
                pltpu.make_async_copy(b0, acc.at[idxs0], sem0).wait()
                ild(sref, i + 2, idxs0, sem0)
                pltpu.make_async_copy(b0, acc.at[idxs1], sem1).wait()
                c3 = jnp.minimum(i + 3, NCHUNKS - 1)
                ild(sref, c3, idxs1, sem1)
                return carry
            lax.fori_loop(0, NPAIR, pair, 0)
            pltpu.sync_copy(b0, acc.at[idxs0], add=True)
            plsc.subcore_barrier()

        for dirn, (gref, sref) in enumerate(
                ((heads_hbm, tails_hbm), (tails_hbm, heads_hbm))):
            s_phase(gref, sref)
            copy_out(S4, 2 * cid + dirn)
        for dirn, sref in enumerate((tails_hbm, heads_hbm)):
            t_phase(sref)
            copy_out(T4, 2 * cid + dirn)
        for dirn, sref in enumerate((tails_hbm, heads_hbm)):
            c_phase(sref)
            copy_out(C4, 2 * cid + dirn)

    return sc_kernel(H, E, heads, tails)


def _tc_finish_body(s4, t4, c4, h, wbig, bf, bb, gam, bet, out):
    s = s4[...]
    t = t4[...]
    c = c4[...]
    Sf = s[0] + s[2]
    Sb = s[1] + s[3]
    Tf = t[0] + t[2]
    Tb = t[1] + t[3]
    X = jnp.concatenate([Sf, Tf, Sb, Tb], axis=1)
    num = jnp.dot(X, wbig[...], preferred_element_type=jnp.float32)
    ct = c[0, :, 0:1] + c[2, :, 0:1]
    ch = c[1, :, 0:1] + c[3, :, 0:1]
    num = num + ct * bf[...] + ch * bb[...]
    agg = num / (ct + ch)
    x = jnp.where(agg >= 0, agg, 0.01 * agg) + h[...]
    mu = jnp.mean(x, axis=1, keepdims=True)
    var = jnp.mean((x - mu) ** 2, axis=1, keepdims=True)
    out[...] = (x - mu) * lax.rsqrt(var + 1e-5) * gam[...] + bet[...]


def _tc_finish(S4, T4, C4, H, Wbig, bf, bb, gam, bet):
    BR = 1024
    grid = NP // BR
    return pl.pallas_call(
        _tc_finish_body,
        grid=(grid,),
        in_specs=[
            pl.BlockSpec((2 * NCORES, BR, D), lambda i: (0, i, 0)),
            pl.BlockSpec((2 * NCORES, BR, D), lambda i: (0, i, 0)),
            pl.BlockSpec((2 * NCORES, BR, D), lambda i: (0, i, 0)),
            pl.BlockSpec((BR, D), lambda i: (i, 0)),
            pl.BlockSpec((4 * D, D), lambda i: (0, 0)),
            pl.BlockSpec((1, D), lambda i: (0, 0)),
            pl.BlockSpec((1, D), lambda i: (0, 0)),
            pl.BlockSpec((1, D), lambda i: (0, 0)),
            pl.BlockSpec((1, D), lambda i: (0, 0)),
        ],
        out_specs=pl.BlockSpec((BR, D), lambda i: (i, 0)),
        out_shape=jax.ShapeDtypeStruct((NP, D), jnp.float32),
    )(S4, T4, C4, H, Wbig, bf, bb, gam, bet)


def kernel(H, E, ht, queries, W_fwd, b_fwd, W_back, b_back, ln_gamma, ln_beta):
    ht32 = ht.astype(jnp.int32)
    S4, T4, C4 = _sc_accumulate(H, E, ht32[:, 0], ht32[:, 1])
    Wbig = jnp.concatenate(
        [W_fwd[:, :D].T, W_fwd[:, D:].T, W_back[:, :D].T, W_back[:, D:].T],
        axis=0)
    Hp = jnp.pad(H, ((0, NP - N_NODES), (0, 0)))
    out = _tc_finish(S4, T4, C4, Hp, Wbig,
                     b_fwd.reshape(1, D), b_back.reshape(1, D),
                     ln_gamma.reshape(1, D), ln_beta.reshape(1, D))
    return out[:N_NODES]
